# baseline (speedup 1.0000x reference)
import functools

import jax
import jax.numpy as jnp
from jax.experimental import pallas as pl
from jax.experimental.pallas import tpu as pltpu

_N_FEATS = 4
_EXPANSION = 4
_RES_SCALE = 0.1


def _stack_kernel(x_ref, w1_ref, w2_ref, out_ref, *, H, W, L, n_pipe):
    HW = H * W
    G = x_ref.shape[0] // n_pipe
    Gm = w1_ref.shape[1]

    col = jax.lax.broadcasted_iota(jnp.int32, (1, HW), 1) % W
    first_col = col == 0
    last_col = col == (W - 1)

    def csh(v, k):
        return pltpu.roll(v, (-k) % HW, axis=1)

    def dw_taps(v):
        vm = jnp.where(first_col, csh(v, W - 1), csh(v, -1))
        vp = jnp.where(last_col, csh(v, -(W - 1)), csh(v, 1))
        return vm, vp

    ones_grp = jnp.concatenate(
        [jnp.ones((1, HW), jnp.bfloat16), jnp.zeros((15, HW), jnp.bfloat16)],
        axis=0)

    def block_step(l, x):
        xb = x.astype(jnp.bfloat16)
        xm, xp = dw_taps(xb)
        x3 = jnp.concatenate([xm, xb, xp], axis=0)
        x3u, x3d = csh(x3, -W), csh(x3, W)
        x3uu = csh(x3, -2 * W)
        x9 = jnp.concatenate([x3u, x3, x3d, ones_grp], axis=0)
        z1 = jnp.dot(w1_ref[l], x9, preferred_element_type=jnp.float32)
        h = jnp.maximum(z1, 0.0).astype(jnp.bfloat16)
        x9u = jnp.concatenate([x3uu, x3u, x3, ones_grp], axis=0)
        z1u = jnp.dot(w1_ref[l], x9u, preferred_element_type=jnp.float32)
        hu = jnp.maximum(z1u, 0.0).astype(jnp.bfloat16)
        hd = csh(h, W)
        h3 = jnp.concatenate([hu, h, hd, ones_grp], axis=0)
        z2 = jnp.dot(w2_ref[l], h3, preferred_element_type=jnp.float32)
        z0, z1c, z2c = z2[0:G], z2[G:2 * G], z2[2 * G:3 * G]
        ym = jnp.where(first_col, csh(z0, W - 1), csh(z0, -1))
        yp = jnp.where(last_col, csh(z2c, -(W - 1)), csh(z2c, 1))
        return x + ym + z1c + yp

    xs = [x_ref[p * G:(p + 1) * G].astype(jnp.float32) for p in range(n_pipe)]
    for l in range(L):
        xs = [block_step(l, x) for x in xs]
    for p in range(n_pipe):
        out_ref[p * G:(p + 1) * G] = xs[p].astype(out_ref.dtype)


def _weight_norm(v, g):
    norm = jnp.sqrt(jnp.sum(v * v, axis=(2, 3, 4), keepdims=True))
    return g[:, :, None, None, None] * v / norm


_NB_PREF = 4
_NPIPE_PREF = 2


def _pick_group(n, c, cmid):
    for nb in range(min(n, _NB_PREF), 0, -1):
        if n % nb == 0 and (nb * c) % 8 == 0 and 9 * nb * c <= 256 \
                and 3 * nb * cmid <= 256:
            return nb
    return 1


def _forward(x_nchw, v1s, g1s, b1s, v2s, g2s, b2s):
    N, C, H, W = x_nchw.shape
    HW = H * W
    L, Cmid = v1s.shape[0], v1s.shape[1]
    Nb = _pick_group(N, C, Cmid)
    G, Gm = Nb * C, Nb * Cmid
    eye = jnp.eye(Nb, dtype=jnp.float32)

    wn1 = _weight_norm(v1s, g1s)
    a1 = jnp.transpose(wn1, (0, 3, 4, 1, 2))
    w1 = jnp.einsum('Ljkab,pq->Lpajkqb', a1, eye).reshape(L, Gm, 9 * G)

    wn2 = _weight_norm(v2s, g2s) * _RES_SCALE
    a2 = jnp.transpose(wn2, (0, 3, 4, 1, 2))
    w2 = jnp.einsum('Ljkab,pq->Lkpajqb', a2, eye).reshape(L, 3 * G, 3 * Gm)

    b1c = jnp.zeros((L, Gm, 16), w1.dtype).at[:, :, 0].set(
        jnp.tile(b1s, (1, Nb)))
    w1 = jnp.concatenate([w1, b1c], axis=2)
    b2c = jnp.zeros((L, 3 * G, 16), w2.dtype).at[:, G:2 * G, 0].set(
        jnp.tile(_RES_SCALE * b2s, (1, Nb)))
    w2 = jnp.concatenate([w2, b2c], axis=2)

    n_pipe = _NPIPE_PREF
    while (N // Nb) % n_pipe != 0:
        n_pipe -= 1
    xf = x_nchw.reshape(N * C, HW)
    kernel_fn = functools.partial(_stack_kernel, H=H, W=W, L=L, n_pipe=n_pipe)
    out = pl.pallas_call(
        kernel_fn,
        out_shape=jax.ShapeDtypeStruct((N * C, HW), x_nchw.dtype),
        grid=(N // (Nb * n_pipe),),
        in_specs=[
            pl.BlockSpec((n_pipe * G, HW), lambda i: (i, 0)),
            pl.BlockSpec((L, Gm, 9 * G + 16), lambda i: (0, 0, 0)),
            pl.BlockSpec((L, 3 * G, 3 * Gm + 16), lambda i: (0, 0, 0)),
        ],
        out_specs=pl.BlockSpec((n_pipe * G, HW), lambda i: (i, 0)),
        compiler_params=pltpu.CompilerParams(
            dimension_semantics=("parallel",),
            vmem_limit_bytes=64 * 2**20),
    )(xf, w1.astype(jnp.bfloat16), w2.astype(jnp.bfloat16))
    return out.reshape(N, C, H, W)


def kernel(x,
           v1_0, g1_0, b1_0, v2_0, g2_0, b2_0,
           v1_1, g1_1, b1_1, v2_1, g2_1, b2_1,
           v1_2, g1_2, b1_2, v2_2, g2_2, b2_2,
           v1_3, g1_3, b1_3, v2_3, g2_3, b2_3,
           v1_4, g1_4, b1_4, v2_4, g2_4, b2_4,
           v1_5, g1_5, b1_5, v2_5, g2_5, b2_5,
           v1_6, g1_6, b1_6, v2_6, g2_6, b2_6,
           v1_7, g1_7, b1_7, v2_7, g2_7, b2_7):
    v1s = jnp.stack([v1_0, v1_1, v1_2, v1_3, v1_4, v1_5, v1_6, v1_7])
    g1s = jnp.stack([g1_0, g1_1, g1_2, g1_3, g1_4, g1_5, g1_6, g1_7])
    b1s = jnp.stack([b1_0, b1_1, b1_2, b1_3, b1_4, b1_5, b1_6, b1_7])
    v2s = jnp.stack([v2_0, v2_1, v2_2, v2_3, v2_4, v2_5, v2_6, v2_7])
    g2s = jnp.stack([g2_0, g2_1, g2_2, g2_3, g2_4, g2_5, g2_6, g2_7])
    b2s = jnp.stack([b2_0, b2_1, b2_2, b2_3, b2_4, b2_5, b2_6, b2_7])
    return _forward(x, v1s, g1s, b1s, v2s, g2s, b2s)


# speedup vs baseline: 1.2878x; 1.2878x over previous
import functools

import jax
import jax.numpy as jnp
from jax.experimental import pallas as pl
from jax.experimental.pallas import tpu as pltpu

_N_FEATS = 4
_EXPANSION = 4
_RES_SCALE = 0.1


def _stack_kernel(x_ref, w1_ref, w2_ref, out_ref, *, H, W, L, n_pipe):
    HW = H * W
    G = x_ref.shape[0] // n_pipe
    Gm = w1_ref.shape[1]

    col = jax.lax.broadcasted_iota(jnp.int32, (1, HW), 1) % W
    first_col = col == 0
    last_col = col == (W - 1)

    def csh(v, k):
        return pltpu.roll(v, (-k) % HW, axis=1)

    def dw_taps(v):
        vm = jnp.where(first_col, csh(v, W - 1), csh(v, -1))
        vp = jnp.where(last_col, csh(v, -(W - 1)), csh(v, 1))
        return vm, vp

    ones_grp = jnp.concatenate(
        [jnp.ones((1, HW), jnp.bfloat16), jnp.zeros((15, HW), jnp.bfloat16)],
        axis=0)

    def block_step(l, x):
        xb = x.astype(jnp.bfloat16)
        xm, xp = dw_taps(xb)
        x3 = jnp.concatenate([xm, xb, xp], axis=0)
        x3u, x3d = csh(x3, -W), csh(x3, W)
        x3uu, x3dd = csh(x3, -2 * W), csh(x3, 2 * W)
        x9 = jnp.concatenate([x3u, x3, x3d, ones_grp], axis=0)
        z1 = jnp.dot(w1_ref[l], x9, preferred_element_type=jnp.float32)
        h = jnp.maximum(z1, 0.0).astype(jnp.bfloat16)
        x9u = jnp.concatenate([x3uu, x3u, x3, ones_grp], axis=0)
        x9d = jnp.concatenate([x3, x3d, x3dd, ones_grp], axis=0)
        z1u = jnp.dot(w1_ref[l], x9u, preferred_element_type=jnp.float32)
        z1d = jnp.dot(w1_ref[l], x9d, preferred_element_type=jnp.float32)
        hu = jnp.maximum(z1u, 0.0).astype(jnp.bfloat16)
        hd = jnp.maximum(z1d, 0.0).astype(jnp.bfloat16)
        h3 = jnp.concatenate([hu, h, hd, ones_grp], axis=0)
        z2 = jnp.dot(w2_ref[l], h3, preferred_element_type=jnp.float32)
        z0, z1c, z2c = z2[0:G], z2[G:2 * G], z2[2 * G:3 * G]
        ym = jnp.where(first_col, csh(z0, W - 1), csh(z0, -1))
        yp = jnp.where(last_col, csh(z2c, -(W - 1)), csh(z2c, 1))
        return x + ym + z1c + yp

    xs = [x_ref[p * G:(p + 1) * G].astype(jnp.float32) for p in range(n_pipe)]
    for l in range(L):
        xs = [block_step(l, x) for x in xs]
    for p in range(n_pipe):
        out_ref[p * G:(p + 1) * G] = xs[p].astype(out_ref.dtype)


def _weight_norm(v, g):
    norm = jnp.sqrt(jnp.sum(v * v, axis=(2, 3, 4), keepdims=True))
    return g[:, :, None, None, None] * v / norm


_NB_PREF = 4
_NPIPE_PREF = 2


def _pick_group(n, c, cmid):
    for nb in range(min(n, _NB_PREF), 0, -1):
        if n % nb == 0 and (nb * c) % 8 == 0 and 9 * nb * c <= 256 \
                and 3 * nb * cmid <= 256:
            return nb
    return 1


def _forward(x_nchw, v1s, g1s, b1s, v2s, g2s, b2s):
    N, C, H, W = x_nchw.shape
    HW = H * W
    L, Cmid = v1s.shape[0], v1s.shape[1]
    Nb = _pick_group(N, C, Cmid)
    G, Gm = Nb * C, Nb * Cmid
    eye = jnp.eye(Nb, dtype=jnp.float32)

    wn1 = _weight_norm(v1s, g1s)
    a1 = jnp.transpose(wn1, (0, 3, 4, 1, 2))
    w1 = jnp.einsum('Ljkab,pq->Lpajkqb', a1, eye).reshape(L, Gm, 9 * G)

    wn2 = _weight_norm(v2s, g2s) * _RES_SCALE
    a2 = jnp.transpose(wn2, (0, 3, 4, 1, 2))
    w2 = jnp.einsum('Ljkab,pq->Lkpajqb', a2, eye).reshape(L, 3 * G, 3 * Gm)

    b1c = jnp.zeros((L, Gm, 16), w1.dtype).at[:, :, 0].set(
        jnp.tile(b1s, (1, Nb)))
    w1 = jnp.concatenate([w1, b1c], axis=2)
    b2c = jnp.zeros((L, 3 * G, 16), w2.dtype).at[:, G:2 * G, 0].set(
        jnp.tile(_RES_SCALE * b2s, (1, Nb)))
    w2 = jnp.concatenate([w2, b2c], axis=2)

    n_pipe = _NPIPE_PREF
    while (N // Nb) % n_pipe != 0:
        n_pipe -= 1
    xf = x_nchw.reshape(N * C, HW)
    kernel_fn = functools.partial(_stack_kernel, H=H, W=W, L=L, n_pipe=n_pipe)
    out = pl.pallas_call(
        kernel_fn,
        out_shape=jax.ShapeDtypeStruct((N * C, HW), x_nchw.dtype),
        grid=(N // (Nb * n_pipe),),
        in_specs=[
            pl.BlockSpec((n_pipe * G, HW), lambda i: (i, 0)),
            pl.BlockSpec((L, Gm, 9 * G + 16), lambda i: (0, 0, 0)),
            pl.BlockSpec((L, 3 * G, 3 * Gm + 16), lambda i: (0, 0, 0)),
        ],
        out_specs=pl.BlockSpec((n_pipe * G, HW), lambda i: (i, 0)),
        compiler_params=pltpu.CompilerParams(
            dimension_semantics=("parallel",),
            vmem_limit_bytes=64 * 2**20),
    )(xf, w1.astype(jnp.bfloat16), w2.astype(jnp.bfloat16))
    return out.reshape(N, C, H, W)


def kernel(x,
           v1_0, g1_0, b1_0, v2_0, g2_0, b2_0,
           v1_1, g1_1, b1_1, v2_1, g2_1, b2_1,
           v1_2, g1_2, b1_2, v2_2, g2_2, b2_2,
           v1_3, g1_3, b1_3, v2_3, g2_3, b2_3,
           v1_4, g1_4, b1_4, v2_4, g2_4, b2_4,
           v1_5, g1_5, b1_5, v2_5, g2_5, b2_5,
           v1_6, g1_6, b1_6, v2_6, g2_6, b2_6,
           v1_7, g1_7, b1_7, v2_7, g2_7, b2_7):
    v1s = jnp.stack([v1_0, v1_1, v1_2, v1_3, v1_4, v1_5, v1_6, v1_7])
    g1s = jnp.stack([g1_0, g1_1, g1_2, g1_3, g1_4, g1_5, g1_6, g1_7])
    b1s = jnp.stack([b1_0, b1_1, b1_2, b1_3, b1_4, b1_5, b1_6, b1_7])
    v2s = jnp.stack([v2_0, v2_1, v2_2, v2_3, v2_4, v2_5, v2_6, v2_7])
    g2s = jnp.stack([g2_0, g2_1, g2_2, g2_3, g2_4, g2_5, g2_6, g2_7])
    b2s = jnp.stack([b2_0, b2_1, b2_2, b2_3, b2_4, b2_5, b2_6, b2_7])
    return _forward(x, v1s, g1s, b1s, v2s, g2s, b2s)
